# baseline (speedup 1.0000x reference)
import dataclasses

import numpy as np

import jax
import jax.numpy as jnp
from jax import lax
from jax.experimental import pallas as pl
from jax.experimental.pallas import tpu as pltpu
from jax.experimental.pallas import tpu_sc as plsc

B, T, FX = 256, 64, 128
INFO = 256
EM_IN = 128
OH_OUT = 64
N_LAYER = 1024
NUM_LAYERS = 4
K = 512

NR = NUM_LAYERS * B
NW = 32
RPW = NR // NW
GRP = 8
NCH = N_LAYER // 16
INT_MIN32 = np.int32(-2147483648)


def _tc_body(xT_ref, e_ref, Wih_ref, Whh_ref, b_ref,
             m1_ref, mb1_ref, m2_ref, mb2_ref,
             g1e_ref, g1t_ref, gb1_ref, g2_ref, gb2_ref, g3_ref, gb3_ref,
             G_ref, out_ref, h_ref, c_ref):
    h_ref[...] = jnp.zeros((B, INFO), jnp.float32)
    c_ref[...] = jnp.zeros((B, INFO), jnp.float32)

    def step(t, carry):
        xt = xT_ref[t]
        gates = (jnp.dot(xt, Wih_ref[...], preferred_element_type=jnp.float32)
                 + jnp.dot(h_ref[...], Whh_ref[...], preferred_element_type=jnp.float32)
                 + b_ref[...])
        i = jax.nn.sigmoid(gates[:, :INFO])
        f = jax.nn.sigmoid(gates[:, INFO:2 * INFO])
        g = jnp.tanh(gates[:, 2 * INFO:3 * INFO])
        o = jax.nn.sigmoid(gates[:, 3 * INFO:])
        c = f * c_ref[...] + i * g
        c_ref[...] = c
        h_ref[...] = o * jnp.tanh(c)
        return carry

    lax.fori_loop(0, T, step, 0)
    traj = h_ref[...]

    emb = (jnp.dot(
        jax.nn.relu(jnp.dot(e_ref[...], m1_ref[...],
                            preferred_element_type=jnp.float32) + mb1_ref[...]),
        m2_ref[...], preferred_element_type=jnp.float32) + mb2_ref[...])

    h1 = jax.nn.relu(
        jnp.dot(emb, g1e_ref[...], preferred_element_type=jnp.float32)
        + jnp.dot(traj, g1t_ref[...], preferred_element_type=jnp.float32)
        + gb1_ref[...])
    h2 = jax.nn.relu(
        jnp.dot(h1, g2_ref[...], preferred_element_type=jnp.float32) + gb2_ref[...])
    mv = jnp.dot(h2, g3_ref[...], preferred_element_type=jnp.float32) + gb3_ref[...]

    for li in range(NUM_LAYERS):
        out_ref[li] = mv[:, li * N_LAYER:(li + 1) * N_LAYER] + G_ref[li]


def _sc_topk_body(z_ref, out_ref, zbuf, ukeys, hist, obuf):
    wid = lax.axis_index("s") * 2 + lax.axis_index("c")
    lanes = lax.iota(jnp.int32, 16)
    ones16 = jnp.ones((16,), jnp.int32)
    zeros16 = jnp.zeros((16,), jnp.int32)
    kvec = jnp.full((16,), K, jnp.int32)

    @pl.loop(0, 2)
    def _task(ti):
        row0 = (wid + ti * NW) * 16
        pltpu.sync_copy(z_ref.at[pl.ds(row0, 16)], zbuf.at[:, pl.ds(0, N_LAYER)])

        def _zero_hist():
            @plsc.parallel_loop(0, 2048, unroll=8)
            def _z(i):
                hist[pl.ds(i * 16, 16)] = zeros16

        _zero_hist()
        @plsc.parallel_loop(0, N_LAYER, unroll=8)
        def _k(j):
            jv = jnp.full((16,), j, jnp.int32)
            v = plsc.load_gather(zbuf, [lanes, jv])
            bts = plsc.bitcast(v, jnp.int32)
            key = bts ^ (lax.shift_right_arithmetic(bts, 31)
                         & jnp.int32(0x7FFFFFFF))
            uk = key ^ INT_MIN32
            ukeys[pl.ds(j * 16, 16)] = uk
            dig = lax.shift_right_logical(uk, 24)
            plsc.addupdate_scatter(
                hist, [(j & 7) * 4096 + dig * 16 + lanes], ones16)

        def _scan(kneed_v):
            @plsc.parallel_loop(
                0, 256, unroll=8,
                carry=(zeros16, zeros16, zeros16,
                       jnp.zeros((16,), jnp.bool_)))
            def _s(i, carry):
                acc, bstar, cgtl, found = carry
                b = 255 - i
                h = hist[pl.ds(b * 16, 16)]
                for c in range(1, 8):
                    h = h + hist[pl.ds(c * 4096 + b * 16, 16)]
                acc2 = acc + h
                hit = jnp.logical_and(jnp.logical_not(found),
                                      acc2 >= kneed_v)
                bstar = jnp.where(hit, b, bstar)
                cgtl = jnp.where(hit, acc, cgtl)
                found = jnp.logical_or(found, acc2 >= kneed_v)
                return acc2, bstar, cgtl, found

            _, bstar, cgtl, _ = _s
            return bstar, cgtl

        bstar, cgtl = _scan(kvec)
        prefix_v = bstar
        kneed_v = kvec - cgtl

        for sbits in (16, 8, 0):
            _zero_hist()

            @plsc.parallel_loop(0, N_LAYER, unroll=8)
            def _l(j, _s_=sbits, _pv_=prefix_v):
                uk = ukeys[pl.ds(j * 16, 16)]
                act = lax.shift_right_logical(uk, _s_ + 8) == _pv_
                dig = (lax.shift_right_logical(uk, _s_)
                       & jnp.int32(0xFF))
                plsc.addupdate_scatter(
                    hist, [(j & 7) * 4096 + dig * 16 + lanes], ones16,
                    mask=act)

            bstar, cgtl = _scan(kneed_v)
            prefix_v = (prefix_v << 8) | bstar
            kneed_v = kneed_v - cgtl

        thr_x = prefix_v ^ INT_MIN32

        @plsc.parallel_loop(0, N_LAYER, unroll=8, carry=zeros16)
        def _f(j, rank):
            uk = ukeys[pl.ds(j * 16, 16)]
            kx = uk ^ INT_MIN32
            gt = kx > thr_x
            eq = uk == prefix_v
            rank = rank + jnp.where(eq, 1, 0)
            sel = jnp.logical_or(gt, jnp.logical_and(eq, rank <= kneed_v))
            val = jnp.where(sel, jnp.float32(1.0), jnp.float32(0.0))
            jv = jnp.full((16,), j, jnp.int32)
            plsc.store_scatter(obuf, [lanes, jv], val)
            return rank
        pltpu.sync_copy(obuf.at[:, pl.ds(0, N_LAYER)], out_ref.at[pl.ds(row0, 16)])


def _sc_topk(z2):
    mesh = plsc.VectorSubcoreMesh(core_axis_name="c", subcore_axis_name="s")
    cp = pltpu.CompilerParams()
    if "needs_layout_passes" in pltpu.CompilerParams.__dataclass_fields__:
        cp = dataclasses.replace(cp, needs_layout_passes=False)
    kern = pl.kernel(
        _sc_topk_body,
        out_type=jax.ShapeDtypeStruct((NR, N_LAYER), jnp.float32),
        mesh=mesh,
        compiler_params=cp,
        scratch_types=[
            pltpu.VMEM((16, N_LAYER + 1), jnp.float32),
            pltpu.VMEM((N_LAYER * 16,), jnp.int32),
            pltpu.VMEM((8 * 4096,), jnp.int32),
            pltpu.VMEM((16, N_LAYER + 1), jnp.float32),
        ],
    )
    return kern(z2)


def kernel(x, embedding_input, W_ih, W_hh, b_ih, b_hh,
           mlp_w1, mlp_b1, mlp_w2, mlp_b2,
           g_w1, g_b1, g_w2, g_b2, g_w3, g_b3):
    xT = jnp.swapaxes(x, 0, 1)
    e = jnp.squeeze(embedding_input, axis=1)
    b = (b_ih + b_hh).reshape(1, 4 * INFO)
    g1e = g_w1[:OH_OUT]
    g1t = g_w1[OH_OUT:]

    gkey = jax.random.key(42)
    G = jnp.stack([
        jax.random.gumbel(jax.random.fold_in(gkey, li), (B, N_LAYER), jnp.float32)
        for li in range(NUM_LAYERS)
    ], axis=0)

    z = pl.pallas_call(
        _tc_body,
        out_shape=jax.ShapeDtypeStruct((NUM_LAYERS, B, N_LAYER), jnp.float32),
        scratch_shapes=[
            pltpu.VMEM((B, INFO), jnp.float32),
            pltpu.VMEM((B, INFO), jnp.float32),
        ],
    )(xT, e, W_ih, W_hh, b,
      mlp_w1, mlp_b1.reshape(1, -1), mlp_w2, mlp_b2.reshape(1, -1),
      g1e, g1t, g_b1.reshape(1, -1), g_w2, g_b2.reshape(1, -1), g_w3,
      g_b3.reshape(1, -1), G)

    masks2 = _sc_topk(z.reshape(NR, N_LAYER))
    return masks2.reshape(NUM_LAYERS, B, N_LAYER)


# speedup vs baseline: 1.0406x; 1.0406x over previous
import dataclasses

import numpy as np

import jax
import jax.numpy as jnp
from jax import lax
from jax.experimental import pallas as pl
from jax.experimental.pallas import tpu as pltpu
from jax.experimental.pallas import tpu_sc as plsc

B, T, FX = 256, 64, 128
INFO = 256
EM_IN = 128
OH_OUT = 64
N_LAYER = 1024
NUM_LAYERS = 4
K = 512

NR = NUM_LAYERS * B
NW = 32
RPW = NR // NW
GRP = 8
NCH = N_LAYER // 16
INT_MIN32 = np.int32(-2147483648)


def _tc_body(xT_ref, e_ref, Wih_ref, Whh_ref, b_ref,
             m1_ref, mb1_ref, m2_ref, mb2_ref,
             g1e_ref, g1t_ref, gb1_ref, g2_ref, gb2_ref, g3_ref, gb3_ref,
             G_ref, out_ref, h_ref, c_ref):
    h_ref[...] = jnp.zeros((B, INFO), jnp.float32)
    c_ref[...] = jnp.zeros((B, INFO), jnp.float32)

    def step(t, carry):
        xt = xT_ref[t]
        gates = (jnp.dot(xt, Wih_ref[...], preferred_element_type=jnp.float32)
                 + jnp.dot(h_ref[...], Whh_ref[...], preferred_element_type=jnp.float32)
                 + b_ref[...])
        i = jax.nn.sigmoid(gates[:, :INFO])
        f = jax.nn.sigmoid(gates[:, INFO:2 * INFO])
        g = jnp.tanh(gates[:, 2 * INFO:3 * INFO])
        o = jax.nn.sigmoid(gates[:, 3 * INFO:])
        c = f * c_ref[...] + i * g
        c_ref[...] = c
        h_ref[...] = o * jnp.tanh(c)
        return carry

    lax.fori_loop(0, T, step, 0)
    traj = h_ref[...]

    emb = (jnp.dot(
        jax.nn.relu(jnp.dot(e_ref[...], m1_ref[...],
                            preferred_element_type=jnp.float32) + mb1_ref[...]),
        m2_ref[...], preferred_element_type=jnp.float32) + mb2_ref[...])

    h1 = jax.nn.relu(
        jnp.dot(emb, g1e_ref[...], preferred_element_type=jnp.float32)
        + jnp.dot(traj, g1t_ref[...], preferred_element_type=jnp.float32)
        + gb1_ref[...])
    h2 = jax.nn.relu(
        jnp.dot(h1, g2_ref[...], preferred_element_type=jnp.float32) + gb2_ref[...])
    mv = jnp.dot(h2, g3_ref[...], preferred_element_type=jnp.float32) + gb3_ref[...]

    for li in range(NUM_LAYERS):
        out_ref[li] = mv[:, li * N_LAYER:(li + 1) * N_LAYER] + G_ref[li]


def _sc_topk_body(z_ref, out_ref, zbuf, ukeys, hist, obuf):
    wid = lax.axis_index("s") * 2 + lax.axis_index("c")
    lanes = lax.iota(jnp.int32, 16)
    ones16 = jnp.ones((16,), jnp.int32)
    zeros16 = jnp.zeros((16,), jnp.int32)
    kvec = jnp.full((16,), K, jnp.int32)

    @pl.loop(0, 2)
    def _task(ti):
        row0 = (wid + ti * NW) * 16
        pltpu.sync_copy(z_ref.at[pl.ds(row0, 16)], zbuf.at[:, pl.ds(0, N_LAYER)])

        def _zero_hist():
            @plsc.parallel_loop(0, 2048, unroll=8)
            def _z(i):
                hist[pl.ds(i * 16, 16)] = zeros16

        _zero_hist()
        @plsc.parallel_loop(0, N_LAYER, unroll=8)
        def _k(j):
            jv = jnp.full((16,), j, jnp.int32)
            v = plsc.load_gather(zbuf, [lanes, jv])
            bts = plsc.bitcast(v, jnp.int32)
            key = bts ^ (lax.shift_right_arithmetic(bts, 31)
                         & jnp.int32(0x7FFFFFFF))
            uk = key ^ INT_MIN32
            ukeys[pl.ds(j * 16, 16)] = uk
            dig = lax.shift_right_logical(uk, 24)
            plsc.addupdate_scatter(
                hist, [(j & 7) * 4096 + dig * 16 + lanes], ones16)

        def _scan(kneed_v):
            @plsc.parallel_loop(
                0, 256, unroll=8,
                carry=(zeros16, zeros16, zeros16,
                       jnp.zeros((16,), jnp.bool_)))
            def _s(i, carry):
                acc, bstar, cgtl, found = carry
                b = 255 - i
                h = hist[pl.ds(b * 16, 16)]
                hist[pl.ds(b * 16, 16)] = zeros16
                for c in range(1, 8):
                    h = h + hist[pl.ds(c * 4096 + b * 16, 16)]
                    hist[pl.ds(c * 4096 + b * 16, 16)] = zeros16
                acc2 = acc + h
                hit = jnp.logical_and(jnp.logical_not(found),
                                      acc2 >= kneed_v)
                bstar = jnp.where(hit, b, bstar)
                cgtl = jnp.where(hit, acc, cgtl)
                found = jnp.logical_or(found, acc2 >= kneed_v)
                return acc2, bstar, cgtl, found

            _, bstar, cgtl, _ = _s
            return bstar, cgtl

        bstar, cgtl = _scan(kvec)
        prefix_v = bstar
        kneed_v = kvec - cgtl

        for sbits in (16, 8, 0):
            @plsc.parallel_loop(0, N_LAYER, unroll=8)
            def _l(j, _s_=sbits, _pv_=prefix_v):
                uk = ukeys[pl.ds(j * 16, 16)]
                act = lax.shift_right_logical(uk, _s_ + 8) == _pv_
                dig = (lax.shift_right_logical(uk, _s_)
                       & jnp.int32(0xFF))
                plsc.addupdate_scatter(
                    hist, [(j & 7) * 4096 + dig * 16 + lanes], ones16,
                    mask=act)

            bstar, cgtl = _scan(kneed_v)
            prefix_v = (prefix_v << 8) | bstar
            kneed_v = kneed_v - cgtl

        thr_x = prefix_v ^ INT_MIN32

        @plsc.parallel_loop(0, N_LAYER, unroll=8, carry=zeros16)
        def _f(j, rank):
            uk = ukeys[pl.ds(j * 16, 16)]
            kx = uk ^ INT_MIN32
            gt = kx > thr_x
            eq = uk == prefix_v
            rank = rank + jnp.where(eq, 1, 0)
            sel = jnp.logical_or(gt, jnp.logical_and(eq, rank <= kneed_v))
            val = jnp.where(sel, jnp.float32(1.0), jnp.float32(0.0))
            jv = jnp.full((16,), j, jnp.int32)
            plsc.store_scatter(obuf, [lanes, jv], val)
            return rank
        pltpu.sync_copy(obuf.at[:, pl.ds(0, N_LAYER)], out_ref.at[pl.ds(row0, 16)])


def _sc_topk(z2):
    mesh = plsc.VectorSubcoreMesh(core_axis_name="c", subcore_axis_name="s")
    cp = pltpu.CompilerParams()
    if "needs_layout_passes" in pltpu.CompilerParams.__dataclass_fields__:
        cp = dataclasses.replace(cp, needs_layout_passes=False)
    kern = pl.kernel(
        _sc_topk_body,
        out_type=jax.ShapeDtypeStruct((NR, N_LAYER), jnp.float32),
        mesh=mesh,
        compiler_params=cp,
        scratch_types=[
            pltpu.VMEM((16, N_LAYER + 1), jnp.float32),
            pltpu.VMEM((N_LAYER * 16,), jnp.int32),
            pltpu.VMEM((8 * 4096,), jnp.int32),
            pltpu.VMEM((16, N_LAYER + 1), jnp.float32),
        ],
    )
    return kern(z2)


def kernel(x, embedding_input, W_ih, W_hh, b_ih, b_hh,
           mlp_w1, mlp_b1, mlp_w2, mlp_b2,
           g_w1, g_b1, g_w2, g_b2, g_w3, g_b3):
    xT = jnp.swapaxes(x, 0, 1)
    e = jnp.squeeze(embedding_input, axis=1)
    b = (b_ih + b_hh).reshape(1, 4 * INFO)
    g1e = g_w1[:OH_OUT]
    g1t = g_w1[OH_OUT:]

    gkey = jax.random.key(42)
    G = jnp.stack([
        jax.random.gumbel(jax.random.fold_in(gkey, li), (B, N_LAYER), jnp.float32)
        for li in range(NUM_LAYERS)
    ], axis=0)

    z = pl.pallas_call(
        _tc_body,
        out_shape=jax.ShapeDtypeStruct((NUM_LAYERS, B, N_LAYER), jnp.float32),
        scratch_shapes=[
            pltpu.VMEM((B, INFO), jnp.float32),
            pltpu.VMEM((B, INFO), jnp.float32),
        ],
    )(xT, e, W_ih, W_hh, b,
      mlp_w1, mlp_b1.reshape(1, -1), mlp_w2, mlp_b2.reshape(1, -1),
      g1e, g1t, g_b1.reshape(1, -1), g_w2, g_b2.reshape(1, -1), g_w3,
      g_b3.reshape(1, -1), G)

    masks2 = _sc_topk(z.reshape(NR, N_LAYER))
    return masks2.reshape(NUM_LAYERS, B, N_LAYER)


# speedup vs baseline: 1.0431x; 1.0023x over previous
import dataclasses

import numpy as np

import jax
import jax.numpy as jnp
from jax import lax
from jax.experimental import pallas as pl
from jax.experimental.pallas import tpu as pltpu
from jax.experimental.pallas import tpu_sc as plsc

B, T, FX = 256, 64, 128
INFO = 256
EM_IN = 128
OH_OUT = 64
N_LAYER = 1024
NUM_LAYERS = 4
K = 512

NR = NUM_LAYERS * B
NW = 32
RPW = NR // NW
GRP = 8
NCH = N_LAYER // 16
INT_MIN32 = np.int32(-2147483648)


def _tc_body(xT_ref, e_ref, Wih_ref, Whh_ref, b_ref,
             m1_ref, mb1_ref, m2_ref, mb2_ref,
             g1e_ref, g1t_ref, gb1_ref, g2_ref, gb2_ref, g3_ref, gb3_ref,
             G_ref, out_ref, h_ref, c_ref):
    h_ref[...] = jnp.zeros((B, INFO), jnp.float32)
    c_ref[...] = jnp.zeros((B, INFO), jnp.float32)

    def step(t, carry):
        xt = xT_ref[t]
        gates = (jnp.dot(xt, Wih_ref[...], preferred_element_type=jnp.float32,
                         precision=lax.Precision.DEFAULT)
                 + jnp.dot(h_ref[...], Whh_ref[...],
                           preferred_element_type=jnp.float32,
                           precision=lax.Precision.DEFAULT)
                 + b_ref[...])
        i = jax.nn.sigmoid(gates[:, :INFO])
        f = jax.nn.sigmoid(gates[:, INFO:2 * INFO])
        g = jnp.tanh(gates[:, 2 * INFO:3 * INFO])
        o = jax.nn.sigmoid(gates[:, 3 * INFO:])
        c = f * c_ref[...] + i * g
        c_ref[...] = c
        h_ref[...] = o * jnp.tanh(c)
        return carry

    lax.fori_loop(0, T, step, 0)
    traj = h_ref[...]

    emb = (jnp.dot(
        jax.nn.relu(jnp.dot(e_ref[...], m1_ref[...],
                            preferred_element_type=jnp.float32) + mb1_ref[...]),
        m2_ref[...], preferred_element_type=jnp.float32) + mb2_ref[...])

    h1 = jax.nn.relu(
        jnp.dot(emb, g1e_ref[...], preferred_element_type=jnp.float32)
        + jnp.dot(traj, g1t_ref[...], preferred_element_type=jnp.float32)
        + gb1_ref[...])
    h2 = jax.nn.relu(
        jnp.dot(h1, g2_ref[...], preferred_element_type=jnp.float32) + gb2_ref[...])
    mv = jnp.dot(h2, g3_ref[...], preferred_element_type=jnp.float32) + gb3_ref[...]

    for li in range(NUM_LAYERS):
        out_ref[li] = mv[:, li * N_LAYER:(li + 1) * N_LAYER] + G_ref[li]


def _sc_topk_body(z_ref, out_ref, zbuf, ukeys, hist, obuf):
    wid = lax.axis_index("s") * 2 + lax.axis_index("c")
    lanes = lax.iota(jnp.int32, 16)
    ones16 = jnp.ones((16,), jnp.int32)
    zeros16 = jnp.zeros((16,), jnp.int32)
    kvec = jnp.full((16,), K, jnp.int32)

    @pl.loop(0, 2)
    def _task(ti):
        row0 = (wid + ti * NW) * 16
        pltpu.sync_copy(z_ref.at[pl.ds(row0, 16)], zbuf.at[:, pl.ds(0, N_LAYER)])

        def _zero_hist():
            @plsc.parallel_loop(0, 2048, unroll=8)
            def _z(i):
                hist[pl.ds(i * 16, 16)] = zeros16

        _zero_hist()
        @plsc.parallel_loop(0, N_LAYER, unroll=8)
        def _k(j):
            jv = jnp.full((16,), j, jnp.int32)
            v = plsc.load_gather(zbuf, [lanes, jv])
            bts = plsc.bitcast(v, jnp.int32)
            key = bts ^ (lax.shift_right_arithmetic(bts, 31)
                         & jnp.int32(0x7FFFFFFF))
            uk = key ^ INT_MIN32
            ukeys[pl.ds(j * 16, 16)] = uk
            dig = lax.shift_right_logical(uk, 24)
            plsc.addupdate_scatter(
                hist, [(j & 7) * 4096 + dig * 16 + lanes], ones16)

        def _scan(kneed_v):
            @plsc.parallel_loop(
                0, 256, unroll=8,
                carry=(zeros16, zeros16, zeros16,
                       jnp.zeros((16,), jnp.bool_)))
            def _s(i, carry):
                acc, bstar, cgtl, found = carry
                b = 255 - i
                h = hist[pl.ds(b * 16, 16)]
                hist[pl.ds(b * 16, 16)] = zeros16
                for c in range(1, 8):
                    h = h + hist[pl.ds(c * 4096 + b * 16, 16)]
                    hist[pl.ds(c * 4096 + b * 16, 16)] = zeros16
                acc2 = acc + h
                hit = jnp.logical_and(jnp.logical_not(found),
                                      acc2 >= kneed_v)
                bstar = jnp.where(hit, b, bstar)
                cgtl = jnp.where(hit, acc, cgtl)
                found = jnp.logical_or(found, acc2 >= kneed_v)
                return acc2, bstar, cgtl, found

            _, bstar, cgtl, _ = _s
            return bstar, cgtl

        bstar, cgtl = _scan(kvec)
        prefix_v = bstar
        kneed_v = kvec - cgtl

        for sbits in (16, 8, 0):
            @plsc.parallel_loop(0, N_LAYER, unroll=8)
            def _l(j, _s_=sbits, _pv_=prefix_v):
                uk = ukeys[pl.ds(j * 16, 16)]
                act = lax.shift_right_logical(uk, _s_ + 8) == _pv_
                dig = (lax.shift_right_logical(uk, _s_)
                       & jnp.int32(0xFF))
                plsc.addupdate_scatter(
                    hist, [(j & 7) * 4096 + dig * 16 + lanes], ones16,
                    mask=act)

            bstar, cgtl = _scan(kneed_v)
            prefix_v = (prefix_v << 8) | bstar
            kneed_v = kneed_v - cgtl

        thr_x = prefix_v ^ INT_MIN32

        @plsc.parallel_loop(0, N_LAYER, unroll=8, carry=zeros16)
        def _f(j, rank):
            uk = ukeys[pl.ds(j * 16, 16)]
            kx = uk ^ INT_MIN32
            gt = kx > thr_x
            eq = uk == prefix_v
            rank = rank + jnp.where(eq, 1, 0)
            sel = jnp.logical_or(gt, jnp.logical_and(eq, rank <= kneed_v))
            val = jnp.where(sel, jnp.float32(1.0), jnp.float32(0.0))
            jv = jnp.full((16,), j, jnp.int32)
            plsc.store_scatter(obuf, [lanes, jv], val)
            return rank
        pltpu.sync_copy(obuf.at[:, pl.ds(0, N_LAYER)], out_ref.at[pl.ds(row0, 16)])


def _sc_topk(z2):
    mesh = plsc.VectorSubcoreMesh(core_axis_name="c", subcore_axis_name="s")
    cp = pltpu.CompilerParams()
    if "needs_layout_passes" in pltpu.CompilerParams.__dataclass_fields__:
        cp = dataclasses.replace(cp, needs_layout_passes=False)
    kern = pl.kernel(
        _sc_topk_body,
        out_type=jax.ShapeDtypeStruct((NR, N_LAYER), jnp.float32),
        mesh=mesh,
        compiler_params=cp,
        scratch_types=[
            pltpu.VMEM((16, N_LAYER + 1), jnp.float32),
            pltpu.VMEM((N_LAYER * 16,), jnp.int32),
            pltpu.VMEM((8 * 4096,), jnp.int32),
            pltpu.VMEM((16, N_LAYER + 1), jnp.float32),
        ],
    )
    return kern(z2)


def kernel(x, embedding_input, W_ih, W_hh, b_ih, b_hh,
           mlp_w1, mlp_b1, mlp_w2, mlp_b2,
           g_w1, g_b1, g_w2, g_b2, g_w3, g_b3):
    xT = jnp.swapaxes(x, 0, 1)
    e = jnp.squeeze(embedding_input, axis=1)
    b = (b_ih + b_hh).reshape(1, 4 * INFO)
    g1e = g_w1[:OH_OUT]
    g1t = g_w1[OH_OUT:]

    gkey = jax.random.key(42)
    G = jnp.stack([
        jax.random.gumbel(jax.random.fold_in(gkey, li), (B, N_LAYER), jnp.float32)
        for li in range(NUM_LAYERS)
    ], axis=0)

    z = pl.pallas_call(
        _tc_body,
        out_shape=jax.ShapeDtypeStruct((NUM_LAYERS, B, N_LAYER), jnp.float32),
        scratch_shapes=[
            pltpu.VMEM((B, INFO), jnp.float32),
            pltpu.VMEM((B, INFO), jnp.float32),
        ],
    )(xT, e, W_ih, W_hh, b,
      mlp_w1, mlp_b1.reshape(1, -1), mlp_w2, mlp_b2.reshape(1, -1),
      g1e, g1t, g_b1.reshape(1, -1), g_w2, g_b2.reshape(1, -1), g_w3,
      g_b3.reshape(1, -1), G)

    masks2 = _sc_topk(z.reshape(NR, N_LAYER))
    return masks2.reshape(NUM_LAYERS, B, N_LAYER)
